# baseline (speedup 1.0000x reference)
import functools

import jax
import jax.numpy as jnp
from jax import lax
from jax.experimental import pallas as pl
from jax.experimental.pallas import tpu as pltpu
from jax.experimental.pallas import tpu_sc as plsc

_DEPTH = 1000
_B, _T = 1024, 200
_ROWS = _B * _T
_NW = 32
_RPW = _ROWS // _NW
_CH = 32
_NCHUNK = _RPW // _CH
_LANE = 16


def _set_vals(buf, xv, off, val):
    for j in range(_CH // _LANE):
        xm = xv[pl.ds(off + j * _LANE, _LANE)] % _DEPTH
        rows = lax.broadcasted_iota(jnp.int32, (_LANE,), 0) + (j * _LANE)
        plsc.store_scatter(buf, [rows, xm], jnp.full((_LANE,), val, jnp.float32))


def _sc_body(x_hbm, out_hbm, xv, b0, b1, s0, s1):
    wid = lax.axis_index("s") * 2 + lax.axis_index("c")
    wbase = wid * _RPW
    pltpu.sync_copy(x_hbm.at[pl.ds(wbase, _RPW)], xv)

    zero16 = jnp.zeros((_LANE,), jnp.float32)
    for buf in (b0, b1):
        def _zrow(r, _, buf=buf):
            for c in range(62):
                buf[r, pl.ds(c * _LANE, _LANE)] = zero16
            buf[r, pl.ds(_DEPTH - _LANE, _LANE)] = zero16
            return 0
        lax.fori_loop(0, _CH, _zrow, 0)

    bufs, sems = (b0, b1), (s0, s1)
    for b in range(2):
        _set_vals(bufs[b], xv, b * _CH, 1.0)
        pltpu.async_copy(bufs[b], out_hbm.at[pl.ds(wbase + b * _CH, _CH)], sems[b])

    def _ring(g, _):
        for b in range(2):
            c = 2 * g + b
            off = c * _CH
            pltpu.make_async_copy(
                bufs[b], out_hbm.at[pl.ds(wbase + off, _CH)], sems[b]
            ).wait()
            _set_vals(bufs[b], xv, off - 2 * _CH, 0.0)
            _set_vals(bufs[b], xv, off, 1.0)
            pltpu.async_copy(bufs[b], out_hbm.at[pl.ds(wbase + off, _CH)], sems[b])
        return 0

    lax.fori_loop(1, _NCHUNK // 2, _ring, 0)

    for b in range(2):
        pltpu.make_async_copy(
            bufs[b], out_hbm.at[pl.ds(wbase, _CH)], sems[b]
        ).wait()


_sc_call = functools.partial(
    pl.kernel,
    out_type=jax.ShapeDtypeStruct((_ROWS, _DEPTH), jnp.float32),
    mesh=plsc.VectorSubcoreMesh(core_axis_name="c", subcore_axis_name="s"),
    scratch_types=[
        pltpu.VMEM((_RPW,), jnp.int32),
        pltpu.VMEM((_CH, _DEPTH), jnp.float32),
        pltpu.VMEM((_CH, _DEPTH), jnp.float32),
        pltpu.SemaphoreType.DMA,
        pltpu.SemaphoreType.DMA,
    ],
    compiler_params=pltpu.CompilerParams(
        use_tc_tiling_on_sc=False, needs_layout_passes=False
    ),
)(_sc_body)


def kernel(x):
    out = _sc_call(jnp.reshape(x, (_ROWS,)))
    return jnp.reshape(out, (_B, _T, _DEPTH))


# speedup vs baseline: 2.0221x; 2.0221x over previous
import functools

import jax
import jax.numpy as jnp
from jax import lax
from jax.experimental import pallas as pl
from jax.experimental.pallas import tpu as pltpu
from jax.experimental.pallas import tpu_sc as plsc

_DEPTH = 1000
_B, _T = 1024, 200
_ROWS = _B * _T
_NW = 32
_RPW = _ROWS // _NW
_CH = 32
_NCHUNK = _RPW // _CH
_LANE = 16


def _set_vals(buf, xv, off, val):
    for j in range(_CH // _LANE):
        xm = xv[pl.ds(off + j * _LANE, _LANE)] % _DEPTH
        rows = lax.broadcasted_iota(jnp.int32, (_LANE,), 0) + (j * _LANE)
        plsc.store_scatter(buf, [rows, xm], jnp.full((_LANE,), val, jnp.float32))


def _sc_body(x_hbm, out_hbm, xv, b0, b1, s0, s1):
    wid = lax.axis_index("s") * 2 + lax.axis_index("c")
    wbase = wid * _RPW
    pltpu.sync_copy(x_hbm.at[pl.ds(wbase, _RPW)], xv)

    zero16 = jnp.zeros((_LANE,), jnp.float32)
    for buf in (b0, b1):
        def _zrow(r, _, buf=buf):
            for c in range(62):
                buf[r, pl.ds(c * _LANE, _LANE)] = zero16
            buf[r, pl.ds(_DEPTH - _LANE, _LANE)] = zero16
            return 0
        lax.fori_loop(0, _CH, _zrow, 0)

    bufs, sems = (b0, b1), (s0, s1)
    for b in range(2):
        _set_vals(bufs[b], xv, b * _CH, 1.0)
        pltpu.async_copy(bufs[b], out_hbm.at[pl.ds(wbase + b * _CH, _CH)], sems[b])

    def _ring(g, _):
        for b in range(2):
            c = 2 * g + b
            off = c * _CH
            pltpu.make_async_copy(
                bufs[b], out_hbm.at[pl.ds(wbase + off, _CH)], sems[b]
            ).wait()
            _set_vals(bufs[b], xv, off - 2 * _CH, 0.0)
            _set_vals(bufs[b], xv, off, 1.0)
            pltpu.async_copy(bufs[b], out_hbm.at[pl.ds(wbase + off, _CH)], sems[b])
        return 0

    lax.fori_loop(1, _NCHUNK // 2, _ring, 0)

    for b in range(2):
        pltpu.make_async_copy(
            bufs[b], out_hbm.at[pl.ds(wbase, _CH)], sems[b]
        ).wait()


_sc_call = functools.partial(
    pl.kernel,
    out_type=jax.ShapeDtypeStruct((_ROWS, _DEPTH), jnp.float32),
    mesh=plsc.VectorSubcoreMesh(core_axis_name="c", subcore_axis_name="s"),
    scratch_types=[
        pltpu.VMEM((_RPW,), jnp.int32),
        pltpu.VMEM((_CH, _DEPTH), jnp.float32),
        pltpu.VMEM((_CH, _DEPTH), jnp.float32),
        pltpu.SemaphoreType.DMA,
        pltpu.SemaphoreType.DMA,
    ],
    compiler_params=pltpu.CompilerParams(needs_layout_passes=False),
)(_sc_body)


def kernel(x):
    out = _sc_call(jnp.reshape(x, (_ROWS,)))
    return jnp.reshape(out, (_B, _T, _DEPTH))
